# Initial kernel scaffold; baseline (speedup 1.0000x reference)
import jax
import jax.numpy as jnp
from jax.experimental import pallas as pl


def kernel(feat, edge_index, W1_w, W1_b, gate_w, gate_b, W2_w, W2_b, bias):
    raise NotImplementedError("write your pallas kernel here")


# speedup vs baseline: 9.5828x; 9.5828x over previous
import functools
import jax
import jax.numpy as jnp
from jax import lax
from jax.experimental import pallas as pl
from jax.experimental.pallas import tpu as pltpu
from jax.experimental.pallas import tpu_sc as plsc

N = 10000
E = 320000
D = 128
EPS = 0.1
NC = 2
NS = 16
NW = NC * NS
EPW = E // NW
C = 80
NCH = EPW // C
NPAD = 10240
RPT = NPAD // NS

f32 = jnp.float32
i32 = jnp.int32

_mesh = plsc.VectorSubcoreMesh(core_axis_name="c", subcore_axis_name="s",
                               num_cores=NC, num_subcores=NS)


@functools.partial(
    pl.kernel,
    out_type=jax.ShapeDtypeStruct((NC, NPAD), f32),
    mesh=_mesh,
    scratch_types=[
        pltpu.VMEM((NCH, C), i32),
        pltpu.VMEM((C,), f32),
        pltpu.VMEM((RPT,), f32),
        pltpu.VMEM_SHARED((NPAD,), f32),
        pltpu.SemaphoreType.DMA,
    ],
    compiler_params=pltpu.CompilerParams(needs_layout_passes=False),
)
def _deg_kernel(dst_hbm, out_hbm, didx, ones_v, zbuf, deg_sh, sem):
    del sem
    cid = lax.axis_index("c")
    sid = lax.axis_index("s")
    wid = sid * NC + cid
    for k in range(C // 16):
        ones_v[pl.ds(k * 16, 16)] = jnp.ones((16,), f32)
    for k in range(RPT // 16):
        zbuf[pl.ds(k * 16, 16)] = jnp.zeros((16,), f32)
    pltpu.sync_copy(zbuf, deg_sh.at[pl.ds(sid * RPT, RPT)])
    pltpu.sync_copy(dst_hbm.at[wid], didx)
    plsc.subcore_barrier()

    def body(c, carry):
        pltpu.sync_copy(ones_v, deg_sh.at[didx.at[c]], add=True)
        return carry

    lax.fori_loop(0, NCH, body, 0)
    plsc.subcore_barrier()
    pltpu.sync_copy(deg_sh.at[pl.ds(sid * RPT, RPT)],
                    out_hbm.at[cid].at[pl.ds(sid * RPT, RPT)])


@functools.partial(
    pl.kernel,
    out_type=jax.ShapeDtypeStruct((NC, NPAD, D), f32),
    mesh=_mesh,
    scratch_types=[
        pltpu.VMEM((N,), f32),
        pltpu.VMEM((N,), f32),
        pltpu.VMEM((N,), f32),
        pltpu.VMEM((C,), i32),
        pltpu.VMEM((C,), i32),
        pltpu.VMEM((C,), f32),
        pltpu.VMEM((C, D), f32),
        pltpu.VMEM((8, D), f32),
        pltpu.VMEM_SHARED((NPAD, D), f32),
        pltpu.SemaphoreType.DMA,
    ],
    compiler_params=pltpu.CompilerParams(needs_layout_passes=False),
)
def _edge_kernel(h_hbm, a_hbm, b_hbm, n_hbm, src_hbm, dst_hbm, out_hbm,
                 av, bv, nv, srow, drow, ebuf, rows, zbuf, zsh, sem):
    cid = lax.axis_index("c")
    sid = lax.axis_index("s")
    wid = sid * NC + cid
    pltpu.sync_copy(a_hbm, av)
    pltpu.sync_copy(b_hbm, bv)
    pltpu.sync_copy(n_hbm, nv)

    for i in range(8):
        for j in range(D // 16):
            zbuf[i, pl.ds(j * 16, 16)] = jnp.zeros((16,), f32)

    def zrow(r, carry):
        pltpu.sync_copy(zbuf, zsh.at[pl.ds(sid * RPT + r * 8, 8)])
        return carry

    lax.fori_loop(0, RPT // 8, zrow, 0)
    plsc.subcore_barrier()

    def mchunk(c, carry):
        pltpu.sync_copy(src_hbm.at[wid].at[c], srow)
        pltpu.sync_copy(dst_hbm.at[wid].at[c], drow)
        cp = pltpu.async_copy(h_hbm.at[srow], rows, sem)
        for k in range(C // 16):
            sv = srow[pl.ds(k * 16, 16)]
            dv = drow[pl.ds(k * 16, 16)]
            x = plsc.load_gather(av, [dv]) + plsc.load_gather(bv, [sv])
            u = jnp.exp(x + x)
            t = 1.0 - 2.0 / (u + 1.0)
            e = t * plsc.load_gather(nv, [dv]) * plsc.load_gather(nv, [sv])
            ebuf[pl.ds(k * 16, 16)] = e
        cp.wait()

        def mrow(i, carry2):
            eb = plsc.load_gather(ebuf, [jnp.broadcast_to(i, (16,))])
            for j in range(D // 16):
                rows[i, pl.ds(j * 16, 16)] = rows[i, pl.ds(j * 16, 16)] * eb
            return carry2

        lax.fori_loop(0, C, mrow, 0)
        pltpu.sync_copy(rows, zsh.at[drow], add=True)
        return carry

    lax.fori_loop(0, NCH, mchunk, 0)
    plsc.subcore_barrier()
    pltpu.sync_copy(zsh.at[pl.ds(sid * RPT, RPT)],
                    out_hbm.at[cid].at[pl.ds(sid * RPT, RPT)])


B = 400
GRID = N // B


def _k1_body(feat_ref, w1t_ref, w1b_ref, g1_ref, g2_ref, gb_ref, deg_ref,
             h_ref, a_ref, b_ref, norm_ref):
    h = jnp.maximum(
        jnp.dot(feat_ref[...], w1t_ref[...], preferred_element_type=f32)
        + w1b_ref[...], 0.0)
    h_ref[...] = h
    a_ref[...] = jnp.sum(h * g1_ref[...], axis=1, keepdims=True) + gb_ref[...]
    b_ref[...] = jnp.sum(h * g2_ref[...], axis=1, keepdims=True)
    d = jnp.sum(deg_ref[...], axis=1, keepdims=True)
    norm_ref[...] = lax.rsqrt(jnp.maximum(d, 1.0))


_k1 = pl.pallas_call(
    _k1_body,
    grid=(GRID,),
    in_specs=[
        pl.BlockSpec((B, D), lambda i: (i, 0)),
        pl.BlockSpec((D, D), lambda i: (0, 0)),
        pl.BlockSpec((1, D), lambda i: (0, 0)),
        pl.BlockSpec((1, D), lambda i: (0, 0)),
        pl.BlockSpec((1, D), lambda i: (0, 0)),
        pl.BlockSpec((1, 1), lambda i: (0, 0)),
        pl.BlockSpec((B, 2), lambda i: (i, 0)),
    ],
    out_specs=[
        pl.BlockSpec((B, D), lambda i: (i, 0)),
        pl.BlockSpec((B, 1), lambda i: (i, 0)),
        pl.BlockSpec((B, 1), lambda i: (i, 0)),
        pl.BlockSpec((B, 1), lambda i: (i, 0)),
    ],
    out_shape=[
        jax.ShapeDtypeStruct((N, D), f32),
        jax.ShapeDtypeStruct((N, 1), f32),
        jax.ShapeDtypeStruct((N, 1), f32),
        jax.ShapeDtypeStruct((N, 1), f32),
    ],
)


def _k2_body(raw_ref, z0_ref, z1_ref, g1_ref, g2_ref, gb_ref,
             h_ref, a_ref, b_ref):
    h = EPS * raw_ref[...] + z0_ref[0] + z1_ref[0]
    h_ref[...] = h
    a_ref[...] = jnp.sum(h * g1_ref[...], axis=1, keepdims=True) + gb_ref[...]
    b_ref[...] = jnp.sum(h * g2_ref[...], axis=1, keepdims=True)


_k2 = pl.pallas_call(
    _k2_body,
    grid=(GRID,),
    in_specs=[
        pl.BlockSpec((B, D), lambda i: (i, 0)),
        pl.BlockSpec((1, B, D), lambda i: (0, i, 0)),
        pl.BlockSpec((1, B, D), lambda i: (1, i, 0)),
        pl.BlockSpec((1, D), lambda i: (0, 0)),
        pl.BlockSpec((1, D), lambda i: (0, 0)),
        pl.BlockSpec((1, 1), lambda i: (0, 0)),
    ],
    out_specs=[
        pl.BlockSpec((B, D), lambda i: (i, 0)),
        pl.BlockSpec((B, 1), lambda i: (i, 0)),
        pl.BlockSpec((B, 1), lambda i: (i, 0)),
    ],
    out_shape=[
        jax.ShapeDtypeStruct((N, D), f32),
        jax.ShapeDtypeStruct((N, 1), f32),
        jax.ShapeDtypeStruct((N, 1), f32),
    ],
)


def _k3_body(raw_ref, z0_ref, z1_ref, w2t_ref, wb_ref, bias_ref, out_ref):
    h = EPS * raw_ref[...] + z0_ref[0] + z1_ref[0]
    out_ref[...] = (jnp.dot(h, w2t_ref[...], preferred_element_type=f32)
                    + wb_ref[...] + bias_ref[...])


_k3 = pl.pallas_call(
    _k3_body,
    grid=(GRID,),
    in_specs=[
        pl.BlockSpec((B, D), lambda i: (i, 0)),
        pl.BlockSpec((1, B, D), lambda i: (0, i, 0)),
        pl.BlockSpec((1, B, D), lambda i: (1, i, 0)),
        pl.BlockSpec((D, D), lambda i: (0, 0)),
        pl.BlockSpec((1, D), lambda i: (0, 0)),
        pl.BlockSpec((1, D), lambda i: (0, 0)),
    ],
    out_specs=pl.BlockSpec((B, D), lambda i: (i, 0)),
    out_shape=jax.ShapeDtypeStruct((N, D), f32),
)


def kernel(feat, edge_index, W1_w, W1_b, gate_w, gate_b, W2_w, W2_b, bias):
    src = edge_index[0].astype(i32).reshape(NW, NCH, C)
    dst = edge_index[1].astype(i32).reshape(NW, NCH, C)

    deg_parts = _deg_kernel(dst)
    deg2 = deg_parts[:, :N].T

    w1t = W1_w.T
    g1 = gate_w[:, :D]
    g2 = gate_w[:, D:]
    gb = gate_b.reshape(1, 1)

    h, a, b, norm = _k1(feat, w1t, W1_b.reshape(1, D), g1, g2, gb, deg2)
    raw = h
    norm1 = norm.reshape(N)

    zp = _edge_kernel(h, a.reshape(N), b.reshape(N), norm1, src, dst)
    h, a, b = _k2(raw, zp, zp, g1, g2, gb)

    zp = _edge_kernel(h, a.reshape(N), b.reshape(N), norm1, src, dst)
    rst = _k3(raw, zp, zp, W2_w.T, W2_b.reshape(1, D), bias.reshape(1, D))
    return rst


# speedup vs baseline: 19.8225x; 2.0685x over previous
import functools
import jax
import jax.numpy as jnp
from jax import lax
from jax.experimental import pallas as pl
from jax.experimental.pallas import tpu as pltpu
from jax.experimental.pallas import tpu_sc as plsc

N = 10000
E = 320000
D = 128
EPS = 0.1
NC = 2
NS = 16
NW = NC * NS
EPW = E // NW
C = 80
NCH = EPW // C
NPAD = 10240
RPT = NPAD // NS

f32 = jnp.float32
i32 = jnp.int32

_mesh = plsc.VectorSubcoreMesh(core_axis_name="c", subcore_axis_name="s",
                               num_cores=NC, num_subcores=NS)


@functools.partial(
    pl.kernel,
    out_type=jax.ShapeDtypeStruct((NC, NPAD), f32),
    mesh=_mesh,
    scratch_types=[
        pltpu.VMEM((NCH, 2, C), i32),
        pltpu.VMEM((C,), f32),
        pltpu.VMEM((RPT,), f32),
        pltpu.VMEM_SHARED((NPAD,), f32),
        pltpu.SemaphoreType.DMA,
    ],
    compiler_params=pltpu.CompilerParams(needs_layout_passes=False),
)
def _deg_kernel(idx_hbm, out_hbm, didx, ones_v, zbuf, deg_sh, sem):
    del sem
    cid = lax.axis_index("c")
    sid = lax.axis_index("s")
    wid = sid * NC + cid
    for k in range(C // 16):
        ones_v[pl.ds(k * 16, 16)] = jnp.ones((16,), f32)
    for k in range(RPT // 16):
        zbuf[pl.ds(k * 16, 16)] = jnp.zeros((16,), f32)
    pltpu.sync_copy(zbuf, deg_sh.at[pl.ds(sid * RPT, RPT)])
    pltpu.sync_copy(idx_hbm.at[wid], didx)
    plsc.subcore_barrier()

    def body(c, carry):
        pltpu.sync_copy(ones_v, deg_sh.at[didx.at[c].at[1]], add=True)
        return carry

    lax.fori_loop(0, NCH, body, 0)
    plsc.subcore_barrier()
    pltpu.sync_copy(deg_sh.at[pl.ds(sid * RPT, RPT)],
                    out_hbm.at[cid].at[pl.ds(sid * RPT, RPT)])


@functools.partial(
    pl.kernel,
    out_type=jax.ShapeDtypeStruct((NC, NPAD, D), f32),
    mesh=_mesh,
    scratch_types=[
        pltpu.VMEM((N,), f32),
        pltpu.VMEM((N,), f32),
        pltpu.VMEM((4, 2, C), i32),
        pltpu.VMEM((C,), f32),
        pltpu.VMEM((2, C, D), f32),
        pltpu.VMEM((8, D), f32),
        pltpu.VMEM_SHARED((NPAD, D), f32),
        pltpu.SemaphoreType.DMA,
        pltpu.SemaphoreType.DMA,
        pltpu.SemaphoreType.DMA,
        pltpu.SemaphoreType.DMA,
    ],
    compiler_params=pltpu.CompilerParams(needs_layout_passes=False),
)
def _edge_kernel(hs_hbm, a_hbm, b_hbm, idx_hbm, out_hbm,
                 av, bv, idx2, ebuf, rows2, zbuf, zsh,
                 sem_i, sem_g0, sem_g1, sem_s):
    cid = lax.axis_index("c")
    sid = lax.axis_index("s")
    wid = sid * NC + cid
    pltpu.sync_copy(a_hbm, av)
    pltpu.sync_copy(b_hbm, bv)

    for i in range(8):
        for j in range(D // 16):
            zbuf[i, pl.ds(j * 16, 16)] = jnp.zeros((16,), f32)

    def zrow(r, carry):
        pltpu.sync_copy(zbuf, zsh.at[pl.ds(sid * RPT + r * 8, 8)])
        return carry

    lax.fori_loop(0, RPT // 8, zrow, 0)
    plsc.subcore_barrier()

    widx = idx_hbm.at[wid]
    pltpu.sync_copy(widx.at[0], idx2.at[0])
    pltpu.async_copy(widx.at[1], idx2.at[1], sem_i)
    pltpu.async_copy(hs_hbm.at[idx2.at[0].at[0]], rows2.at[0], sem_g0)

    def mchunk(c, carry):
        p4 = jnp.bitwise_and(c, 3)
        p2 = jnp.bitwise_and(c, 1)
        q2 = 1 - p2
        for k in range(C // 16):
            sv = idx2[p4, 0, pl.ds(k * 16, 16)]
            dv = idx2[p4, 1, pl.ds(k * 16, 16)]
            x = plsc.load_gather(av, [dv]) + plsc.load_gather(bv, [sv])
            u = jnp.exp(x + x)
            ebuf[pl.ds(k * 16, 16)] = 1.0 - 2.0 / (u + 1.0)

        @pl.when(c >= 1)
        def _():
            pltpu.make_async_copy(hs_hbm.at[pl.ds(0, C)], rows2.at[0],
                                  sem_s).wait()

        @pl.when(c < NCH - 1)
        def _():
            pltpu.make_async_copy(widx.at[0], idx2.at[0], sem_i).wait()

            @pl.when(c < NCH - 2)
            def _():
                pltpu.async_copy(widx.at[c + 2],
                                 idx2.at[jnp.bitwise_and(c + 2, 3)], sem_i)

            qp4 = jnp.bitwise_and(c + 1, 3)

            @pl.when(q2 == 0)
            def _():
                pltpu.async_copy(hs_hbm.at[idx2.at[qp4].at[0]], rows2.at[q2],
                                 sem_g0)

            @pl.when(q2 == 1)
            def _():
                pltpu.async_copy(hs_hbm.at[idx2.at[qp4].at[0]], rows2.at[q2],
                                 sem_g1)

        @pl.when(p2 == 0)
        def _():
            pltpu.make_async_copy(hs_hbm.at[pl.ds(0, C)], rows2.at[0],
                                  sem_g0).wait()

        @pl.when(p2 == 1)
        def _():
            pltpu.make_async_copy(hs_hbm.at[pl.ds(0, C)], rows2.at[0],
                                  sem_g1).wait()

        def mrow(i2, carry2):
            i = i2 * 2
            for u in range(2):
                eb = plsc.load_gather(ebuf, [jnp.broadcast_to(i + u, (16,))])
                for j in range(D // 16):
                    rows2[p2, i + u, pl.ds(j * 16, 16)] = (
                        rows2[p2, i + u, pl.ds(j * 16, 16)] * eb)
            return carry2

        lax.fori_loop(0, C // 2, mrow, 0)
        pltpu.async_copy(rows2.at[p2], zsh.at[idx2.at[p4].at[1]], sem_s,
                         add=True)
        return carry

    lax.fori_loop(0, NCH, mchunk, 0)
    pltpu.make_async_copy(hs_hbm.at[pl.ds(0, C)], rows2.at[0], sem_s).wait()
    plsc.subcore_barrier()
    pltpu.sync_copy(zsh.at[pl.ds(sid * RPT, RPT)],
                    out_hbm.at[cid].at[pl.ds(sid * RPT, RPT)])


B = 400
GRID = N // B


def _k1_body(feat_ref, w1t_ref, w1b_ref, g1_ref, g2_ref, gb_ref, deg_ref,
             h_ref, hs_ref, a_ref, b_ref, norm_ref):
    h = jnp.maximum(
        jnp.dot(feat_ref[...], w1t_ref[...], preferred_element_type=f32)
        + w1b_ref[...], 0.0)
    h_ref[...] = h
    a_ref[...] = jnp.sum(h * g1_ref[...], axis=1, keepdims=True) + gb_ref[...]
    b_ref[...] = jnp.sum(h * g2_ref[...], axis=1, keepdims=True)
    d = jnp.sum(deg_ref[...], axis=1, keepdims=True)
    nrm = lax.rsqrt(jnp.maximum(d, 1.0))
    norm_ref[...] = nrm
    hs_ref[...] = h * nrm


_k1 = pl.pallas_call(
    _k1_body,
    grid=(GRID,),
    in_specs=[
        pl.BlockSpec((B, D), lambda i: (i, 0)),
        pl.BlockSpec((D, D), lambda i: (0, 0)),
        pl.BlockSpec((1, D), lambda i: (0, 0)),
        pl.BlockSpec((1, D), lambda i: (0, 0)),
        pl.BlockSpec((1, D), lambda i: (0, 0)),
        pl.BlockSpec((1, 1), lambda i: (0, 0)),
        pl.BlockSpec((B, 2), lambda i: (i, 0)),
    ],
    out_specs=[
        pl.BlockSpec((B, D), lambda i: (i, 0)),
        pl.BlockSpec((B, D), lambda i: (i, 0)),
        pl.BlockSpec((B, 1), lambda i: (i, 0)),
        pl.BlockSpec((B, 1), lambda i: (i, 0)),
        pl.BlockSpec((B, 1), lambda i: (i, 0)),
    ],
    out_shape=[
        jax.ShapeDtypeStruct((N, D), f32),
        jax.ShapeDtypeStruct((N, D), f32),
        jax.ShapeDtypeStruct((N, 1), f32),
        jax.ShapeDtypeStruct((N, 1), f32),
        jax.ShapeDtypeStruct((N, 1), f32),
    ],
)


def _k2_body(raw_ref, z0_ref, z1_ref, norm_ref, g1_ref, g2_ref, gb_ref,
             hs_ref, a_ref, b_ref):
    nrm = norm_ref[...]
    h = EPS * raw_ref[...] + nrm * (z0_ref[0] + z1_ref[0])
    hs_ref[...] = h * nrm
    a_ref[...] = jnp.sum(h * g1_ref[...], axis=1, keepdims=True) + gb_ref[...]
    b_ref[...] = jnp.sum(h * g2_ref[...], axis=1, keepdims=True)


_k2 = pl.pallas_call(
    _k2_body,
    grid=(GRID,),
    in_specs=[
        pl.BlockSpec((B, D), lambda i: (i, 0)),
        pl.BlockSpec((1, B, D), lambda i: (0, i, 0)),
        pl.BlockSpec((1, B, D), lambda i: (1, i, 0)),
        pl.BlockSpec((B, 1), lambda i: (i, 0)),
        pl.BlockSpec((1, D), lambda i: (0, 0)),
        pl.BlockSpec((1, D), lambda i: (0, 0)),
        pl.BlockSpec((1, 1), lambda i: (0, 0)),
    ],
    out_specs=[
        pl.BlockSpec((B, D), lambda i: (i, 0)),
        pl.BlockSpec((B, 1), lambda i: (i, 0)),
        pl.BlockSpec((B, 1), lambda i: (i, 0)),
    ],
    out_shape=[
        jax.ShapeDtypeStruct((N, D), f32),
        jax.ShapeDtypeStruct((N, 1), f32),
        jax.ShapeDtypeStruct((N, 1), f32),
    ],
)


def _k3_body(raw_ref, z0_ref, z1_ref, norm_ref, w2t_ref, wb_ref, bias_ref,
             out_ref):
    h = EPS * raw_ref[...] + norm_ref[...] * (z0_ref[0] + z1_ref[0])
    out_ref[...] = (jnp.dot(h, w2t_ref[...], preferred_element_type=f32)
                    + wb_ref[...] + bias_ref[...])


_k3 = pl.pallas_call(
    _k3_body,
    grid=(GRID,),
    in_specs=[
        pl.BlockSpec((B, D), lambda i: (i, 0)),
        pl.BlockSpec((1, B, D), lambda i: (0, i, 0)),
        pl.BlockSpec((1, B, D), lambda i: (1, i, 0)),
        pl.BlockSpec((B, 1), lambda i: (i, 0)),
        pl.BlockSpec((D, D), lambda i: (0, 0)),
        pl.BlockSpec((1, D), lambda i: (0, 0)),
        pl.BlockSpec((1, D), lambda i: (0, 0)),
    ],
    out_specs=pl.BlockSpec((B, D), lambda i: (i, 0)),
    out_shape=jax.ShapeDtypeStruct((N, D), f32),
)


def kernel(feat, edge_index, W1_w, W1_b, gate_w, gate_b, W2_w, W2_b, bias):
    ei = edge_index.astype(i32)
    packed = ei.reshape(2, NW, NCH, C).transpose(1, 2, 0, 3)

    deg_parts = _deg_kernel(packed)
    deg2 = deg_parts[:, :N].T

    w1t = W1_w.T
    g1 = gate_w[:, :D]
    g2 = gate_w[:, D:]
    gb = gate_b.reshape(1, 1)

    raw, hs, a, b, norm = _k1(feat, w1t, W1_b.reshape(1, D), g1, g2, gb, deg2)

    zp = _edge_kernel(hs, a.reshape(N), b.reshape(N), packed)
    hs, a, b = _k2(raw, zp, zp, norm, g1, g2, gb)

    zp = _edge_kernel(hs, a.reshape(N), b.reshape(N), packed)
    rst = _k3(raw, zp, zp, norm, W2_w.T, W2_b.reshape(1, D),
              bias.reshape(1, D))
    return rst
